# baseline (speedup 1.0000x reference)
import functools

import jax
import jax.numpy as jnp
from jax import lax
from jax.experimental import pallas as pl
from jax.experimental.pallas import tpu as pltpu
from jax.experimental.pallas import tpu_sc as plsc

F32 = jnp.float32
I32 = jnp.int32
D = 256
DH = 128
CHUNK = 128
BM = 512


def _mesh():
    return plsc.VectorSubcoreMesh(core_axis_name="c", subcore_axis_name="s")


def _zero_rows(ref, nrows, ngroups=8):
    z = jnp.zeros((16,), F32)

    def row(r, carry):
        for u in range(ngroups):
            ref[r, pl.ds(u * 16, 16)] = z
        return carry

    lax.fori_loop(0, nrows, row, 0)


def _zero_flat(ref, nvec):
    z = jnp.zeros((16,), F32)

    def it(i, carry):
        off = pl.multiple_of(i * 16, 16)
        ref[pl.ds(off, 16)] = z
        return carry

    lax.fori_loop(0, nvec, it, 0)


def _reduce_partials(s, c, part_v, buf_sh, tmp_v, acc_v, out_hbm, Np):
    Q = Np // 16
    pltpu.sync_copy(part_v, buf_sh.at[s])
    plsc.subcore_barrier()
    q0 = pl.multiple_of(s * Q, 8)
    pltpu.sync_copy(buf_sh.at[0, pl.ds(q0, Q)], acc_v)

    def addone(t, carry):
        pltpu.sync_copy(buf_sh.at[t, pl.ds(q0, Q)], tmp_v)

        def vadd(i, cc):
            off = pl.multiple_of(i * 16, 16)
            acc_v[pl.ds(off, 16)] = (acc_v[pl.ds(off, 16)]
                                     + tmp_v[pl.ds(off, 16)])
            return cc

        lax.fori_loop(0, Q // 16, vadd, 0)
        return carry

    lax.fori_loop(1, 16, addone, 0)
    pltpu.sync_copy(acc_v, out_hbm.at[c, pl.ds(q0, Q)])


@functools.lru_cache(None)
def _deg_kernel(Epad, Np):
    epw = Epad // 32
    ng = epw // 16
    Q = Np // 16

    @functools.partial(
        pl.kernel,
        out_type=jax.ShapeDtypeStruct((2, Np), F32),
        mesh=_mesh(),
        compiler_params=pltpu.CompilerParams(needs_layout_passes=False),
        scratch_types=[
            pltpu.VMEM((epw,), I32),
            pltpu.VMEM((Np,), F32),
            pltpu.VMEM((Q,), F32),
            pltpu.VMEM((Q,), F32),
            pltpu.VMEM_SHARED((16, Np), F32),
        ],
    )
    def k(dst_hbm, out_hbm, dst_v, part_v, tmp_v, acc_v, buf_sh):
        c = lax.axis_index("c")
        s = lax.axis_index("s")
        wid = s * 2 + c
        base = wid * epw
        pltpu.sync_copy(dst_hbm.at[pl.ds(base, epw)], dst_v)
        _zero_flat(part_v, Np // 16)
        ones = jnp.ones((16,), F32)

        def body(i, carry):
            off = pl.multiple_of(i * 16, 16)
            dv = dst_v[pl.ds(off, 16)]
            plsc.addupdate_scatter(part_v, [dv], ones)
            return carry

        lax.fori_loop(0, ng, body, 0)
        _reduce_partials(s, c, part_v, buf_sh, tmp_v, acc_v, out_hbm, Np)

    return k


@functools.lru_cache(None)
def _gat_edge_kernel(Epad, Np):
    epw = Epad // 32
    ng = epw // 16
    Q = Np // 16

    @functools.partial(
        pl.kernel,
        out_type=(
            jax.ShapeDtypeStruct((Epad,), F32),
            jax.ShapeDtypeStruct((2, Np), F32),
        ),
        mesh=_mesh(),
        compiler_params=pltpu.CompilerParams(needs_layout_passes=False),
        scratch_types=[
            pltpu.VMEM((epw,), I32),
            pltpu.VMEM((epw,), I32),
            pltpu.VMEM((epw,), F32),
            pltpu.VMEM((Np,), F32),
            pltpu.VMEM((Np,), F32),
            pltpu.VMEM((Np,), F32),
            pltpu.VMEM((Np,), F32),
            pltpu.VMEM((Q,), F32),
            pltpu.VMEM((Q,), F32),
            pltpu.VMEM_SHARED((16, Np), F32),
        ],
    )
    def k(src_hbm, dst_hbm, alpha_hbm, beta_hbm, cmax_hbm,
          ee_hbm, den_hbm,
          src_v, dst_v, ee_v, alpha_v, beta_v, cmax_v, part_v, tmp_v, acc_v,
          buf_sh):
        c = lax.axis_index("c")
        s = lax.axis_index("s")
        wid = s * 2 + c
        base = wid * epw
        pltpu.sync_copy(src_hbm.at[pl.ds(base, epw)], src_v)
        pltpu.sync_copy(dst_hbm.at[pl.ds(base, epw)], dst_v)
        pltpu.sync_copy(alpha_hbm, alpha_v)
        pltpu.sync_copy(beta_hbm, beta_v)
        pltpu.sync_copy(cmax_hbm, cmax_v)
        _zero_flat(part_v, Np // 16)

        def body(i, carry):
            off = pl.multiple_of(i * 16, 16)
            sv = src_v[pl.ds(off, 16)]
            dv = dst_v[pl.ds(off, 16)]
            a = plsc.load_gather(alpha_v, [sv])
            b = plsc.load_gather(beta_v, [dv])
            cm = plsc.load_gather(cmax_v, [dv])
            e = a + b
            e = jnp.where(e >= 0.0, e, 0.2 * e)
            ee = jnp.exp(e - cm)
            ee_v[pl.ds(off, 16)] = ee
            plsc.addupdate_scatter(part_v, [dv], ee)
            return carry

        lax.fori_loop(0, ng, body, 0)
        pltpu.sync_copy(ee_v, ee_hbm.at[pl.ds(base, epw)])
        _reduce_partials(s, c, part_v, buf_sh, tmp_v, acc_v, den_hbm, Np)

    return k


@functools.lru_cache(None)
def _edge_agg_kernel(Epad, Np, weighted, ncol):
    ept = Epad // 16
    nch = ept // CHUNK
    rpt = Np // 16
    ntab = D // ncol

    scratch = [
        pltpu.VMEM((nch, CHUNK), I32),
        pltpu.VMEM((nch, CHUNK), I32),
        pltpu.VMEM((CHUNK, ncol), F32),
        pltpu.VMEM((CHUNK, ncol), F32),
        pltpu.VMEM((CHUNK, ncol), F32),
        pltpu.VMEM((CHUNK, ncol), F32),
        pltpu.VMEM((CHUNK,), I32),
        pltpu.VMEM_SHARED((Np, ncol), F32),
        pltpu.SemaphoreType.DMA,
        pltpu.SemaphoreType.DMA,
        pltpu.SemaphoreType.DMA,
        pltpu.SemaphoreType.DMA,
    ]
    if weighted:
        scratch.insert(2, pltpu.VMEM((nch, CHUNK), F32))

    def body(*refs):
        tabs = refs[:ntab]
        refs = refs[ntab:]
        if weighted:
            (src2_hbm, dst2_hbm, ew2_hbm, out_hbm,
             src2_v, dst2_v, ew2_v, g0b, g1b, s0b, s1b, didx_v, acc_sh,
             g0s, g1s, s0s, s1s) = refs
        else:
            (src2_hbm, dst2_hbm, out_hbm,
             src2_v, dst2_v, g0b, g1b, s0b, s1b, didx_v, acc_sh,
             g0s, g1s, s0s, s1s) = refs
            ew2_v = None
        c = lax.axis_index("c")
        s = lax.axis_index("s")
        row0 = s * nch
        pltpu.sync_copy(src2_hbm.at[pl.ds(row0, nch)], src2_v)
        pltpu.sync_copy(dst2_hbm.at[pl.ds(row0, nch)], dst2_v)
        if weighted:
            pltpu.sync_copy(ew2_hbm.at[pl.ds(row0, nch)], ew2_v)

        def process(i, gbuf, sbuf):
            def rowgrp(g, cc):
                goff = pl.multiple_of(g * 16, 16)
                if weighted:
                    wvec = ew2_v[i, pl.ds(goff, 16)]
                for j in range(16):
                    r = goff + j
                    for u in range(ncol // 16):
                        v = gbuf[r, pl.ds(u * 16, 16)]
                        if weighted:
                            v = v * wvec[j]
                        sbuf[r, pl.ds(u * 16, 16)] = v
                return cc

            lax.fori_loop(0, CHUNK // 16, rowgrp, 0)

        def run(tbl, oslot):
            def gstart(i, buf, sem):
                del sem
                pltpu.sync_copy(tbl.at[src2_v.at[i]], buf)

            def gwait(i, buf, sem):
                del i, buf, sem

            def sdo(i, buf):
                for u in range(CHUNK // 16):
                    didx_v[pl.ds(u * 16, 16)] = dst2_v[i, pl.ds(u * 16, 16)]
                pltpu.sync_copy(buf, acc_sh.at[didx_v], add=True)

            _zero_rows(s0b, CHUNK, ncol // 16)

            def zacc(j, cc):
                r0 = pl.multiple_of(s * rpt + j * CHUNK, 8)
                pltpu.sync_copy(s0b, acc_sh.at[pl.ds(r0, CHUNK)])
                return cc

            lax.fori_loop(0, rpt // CHUNK, zacc, 0)
            gstart(0, g0b, g0s)
            gstart(1, g1b, g1s)
            plsc.subcore_barrier()
            gwait(0, g0b, g0s)
            process(0, g0b, s0b)
            gstart(2, g0b, g0s)
            sdo(0, s0b)
            gwait(1, g1b, g1s)
            process(1, g1b, s1b)
            gstart(3, g1b, g1s)
            sdo(1, s1b)

            def pair(j, cc):
                i0 = 2 * j
                gwait(i0, g0b, g0s)
                process(i0, g0b, s0b)
                gstart(i0 + 2, g0b, g0s)
                sdo(i0, s0b)
                i1 = i0 + 1
                gwait(i1, g1b, g1s)
                process(i1, g1b, s1b)
                gstart(i1 + 2, g1b, g1s)
                sdo(i1, s1b)
                return cc

            lax.fori_loop(1, nch // 2 - 1, pair, 0)
            i0 = nch - 2
            gwait(i0, g0b, g0s)
            process(i0, g0b, s0b)
            sdo(i0, s0b)
            i1 = nch - 1
            gwait(i1, g1b, g1s)
            process(i1, g1b, s1b)
            sdo(i1, s1b)
            plsc.subcore_barrier()

            def wout(j, cc):
                r0 = pl.multiple_of(s * rpt + j * CHUNK, 8)
                pltpu.sync_copy(acc_sh.at[pl.ds(r0, CHUNK)], g0b)
                pltpu.sync_copy(g0b, out_hbm.at[oslot, pl.ds(r0, CHUNK)])
                return cc

            lax.fori_loop(0, rpt // CHUNK, wout, 0)
            plsc.subcore_barrier()

        @pl.when(c == 0)
        def _():
            for q in range(0, ntab, 2):
                run(tabs[q], q)

        @pl.when(c == 1)
        def _():
            for q in range(1, ntab, 2):
                run(tabs[q], q)

    return pl.kernel(
        body,
        out_type=jax.ShapeDtypeStruct((ntab, Np, ncol), F32),
        mesh=_mesh(),
        compiler_params=pltpu.CompilerParams(
            needs_layout_passes=False,
            use_tc_tiling_on_sc=(ncol == 128)),
        scratch_types=scratch,
    )


def _grid(N):
    return (N + BM - 1) // BM


@functools.lru_cache(None)
def _sage_post(N):
    def body(agg_ref, d0_ref, d1_ref, x_ref, wn_ref, bn_ref, ws_ref, o_ref):
        deg = d0_ref[...] + d1_ref[...]
        deg = jnp.maximum(deg, 1.0)
        a = agg_ref[...] / deg
        o_ref[...] = (jnp.dot(a, wn_ref[...], preferred_element_type=F32)
                      + bn_ref[...]
                      + jnp.dot(x_ref[...], ws_ref[...],
                                preferred_element_type=F32))

    return pl.pallas_call(
        body,
        grid=(_grid(N),),
        in_specs=[
            pl.BlockSpec((BM, D), lambda i: (i, 0)),
            pl.BlockSpec((BM, 1), lambda i: (i, 0)),
            pl.BlockSpec((BM, 1), lambda i: (i, 0)),
            pl.BlockSpec((BM, D), lambda i: (i, 0)),
            pl.BlockSpec((D, D), lambda i: (0, 0)),
            pl.BlockSpec((1, D), lambda i: (0, 0)),
            pl.BlockSpec((D, D), lambda i: (0, 0)),
        ],
        out_specs=pl.BlockSpec((BM, D), lambda i: (i, 0)),
        out_shape=jax.ShapeDtypeStruct((N, D), F32),
    )


@functools.lru_cache(None)
def _gat_pre(N):
    def body(y_ref, wg_ref, asrc_ref, adst_ref, h_ref, al_ref, be_ref):
        h = jnp.dot(y_ref[...], wg_ref[...], preferred_element_type=F32)
        h_ref[...] = h
        al_ref[...] = jnp.dot(h, asrc_ref[...], preferred_element_type=F32)
        be_ref[...] = jnp.dot(h, adst_ref[...], preferred_element_type=F32)

    return pl.pallas_call(
        body,
        grid=(_grid(N),),
        in_specs=[
            pl.BlockSpec((BM, D), lambda i: (i, 0)),
            pl.BlockSpec((D, D), lambda i: (0, 0)),
            pl.BlockSpec((D, 1), lambda i: (0, 0)),
            pl.BlockSpec((D, 1), lambda i: (0, 0)),
        ],
        out_specs=[
            pl.BlockSpec((BM, D), lambda i: (i, 0)),
            pl.BlockSpec((BM, 1), lambda i: (i, 0)),
            pl.BlockSpec((BM, 1), lambda i: (i, 0)),
        ],
        out_shape=[
            jax.ShapeDtypeStruct((N, D), F32),
            jax.ShapeDtypeStruct((N, 1), F32),
            jax.ShapeDtypeStruct((N, 1), F32),
        ],
    )


@functools.lru_cache(None)
def _gat_scalar(N):
    def body(al_ref, be_ref, cm_ref, el_ref):
        al = al_ref[...]
        be = be_ref[...]
        amax = jnp.max(al)
        t = amax + be
        cm = jnp.where(t >= 0.0, t, 0.2 * t)
        u = al + be
        u = jnp.where(u >= 0.0, u, 0.2 * u)
        cm_ref[...] = cm
        el_ref[...] = jnp.exp(u - cm)

    return pl.pallas_call(
        body,
        grid=(1,),
        in_specs=[
            pl.BlockSpec((N, 1), lambda i: (0, 0)),
            pl.BlockSpec((N, 1), lambda i: (0, 0)),
        ],
        out_specs=[
            pl.BlockSpec((N, 1), lambda i: (0, 0)),
            pl.BlockSpec((N, 1), lambda i: (0, 0)),
        ],
        out_shape=[
            jax.ShapeDtypeStruct((N, 1), F32),
            jax.ShapeDtypeStruct((N, 1), F32),
        ],
    )


@functools.lru_cache(None)
def _gat_post(N):
    def body(nu_ref, d0_ref, d1_ref, el_ref, h_ref, y_ref, wr_ref, bg_ref,
             o_ref):
        el = el_ref[...]
        den = d0_ref[...] + d1_ref[...] + el + 1e-16
        o_ref[...] = ((nu_ref[...] + h_ref[...] * el) / den
                      + bg_ref[...]
                      + jnp.dot(y_ref[...], wr_ref[...],
                                preferred_element_type=F32))

    return pl.pallas_call(
        body,
        grid=(_grid(N),),
        in_specs=[
            pl.BlockSpec((BM, D), lambda i: (i, 0)),
            pl.BlockSpec((BM, 1), lambda i: (i, 0)),
            pl.BlockSpec((BM, 1), lambda i: (i, 0)),
            pl.BlockSpec((BM, 1), lambda i: (i, 0)),
            pl.BlockSpec((BM, D), lambda i: (i, 0)),
            pl.BlockSpec((BM, D), lambda i: (i, 0)),
            pl.BlockSpec((D, D), lambda i: (0, 0)),
            pl.BlockSpec((1, D), lambda i: (0, 0)),
        ],
        out_specs=pl.BlockSpec((BM, D), lambda i: (i, 0)),
        out_shape=jax.ShapeDtypeStruct((N, D), F32),
    )


@functools.lru_cache(None)
def _dec1(M, K):
    def body(a_ref, b_ref, o_ref):
        o_ref[...] = jnp.dot(a_ref[...], b_ref[...],
                             preferred_element_type=F32)

    return pl.pallas_call(
        body,
        grid=(_grid(M),),
        in_specs=[
            pl.BlockSpec((BM, K), lambda i: (i, 0)),
            pl.BlockSpec((K, K), lambda i: (0, 0)),
        ],
        out_specs=pl.BlockSpec((BM, K), lambda i: (i, 0)),
        out_shape=jax.ShapeDtypeStruct((M, K), F32),
    )


@functools.lru_cache(None)
def _dec2(M, Nc, K):
    def body(t_ref, d_ref, o_ref):
        prod = lax.dot_general(t_ref[...], d_ref[...],
                               (((1,), (1,)), ((), ())),
                               preferred_element_type=F32)
        o_ref[...] = jax.nn.sigmoid(prod)

    return pl.pallas_call(
        body,
        grid=(_grid(M), _grid(Nc)),
        in_specs=[
            pl.BlockSpec((BM, K), lambda i, j: (i, 0)),
            pl.BlockSpec((BM, K), lambda i, j: (j, 0)),
        ],
        out_specs=pl.BlockSpec((BM, BM), lambda i, j: (i, j)),
        out_shape=jax.ShapeDtypeStruct((M, Nc), F32),
    )


def _sc_deg(dstp, Epad, Np):
    return _deg_kernel(Epad, Np)(dstp)


def _sc_gat_edge(srcp, dstp, alpha, beta, cmax, Epad, Np):
    return _gat_edge_kernel(Epad, Np)(srcp, dstp, alpha, beta, cmax)


def _sc_edge_agg(tabs, srcp, dstp, ew, Epad, Np, ncol):
    src2 = srcp.reshape(Epad // CHUNK, CHUNK)
    dst2 = dstp.reshape(Epad // CHUNK, CHUNK)
    if ew is None:
        return _edge_agg_kernel(Epad, Np, False, ncol)(*tabs, src2, dst2)
    return _edge_agg_kernel(Epad, Np, True, ncol)(
        *tabs, src2, dst2, ew.reshape(Epad // CHUNK, CHUNK))


def _pad_split(v, Np, ncol):
    n = v.shape[0]
    vp = jnp.concatenate([v, jnp.zeros((Np - n, D), F32)], axis=0)
    return tuple(vp[:, ncol * q:ncol * (q + 1)] for q in range(D // ncol))


def _pad1(v, Np):
    v = v.reshape(-1)
    return jnp.concatenate([v, jnp.zeros((Np - v.shape[0],), F32)])


def _branch(x, edge_idx, layers):
    n = x.shape[0]
    e = edge_idx.shape[1]
    Epad = ((e + 4095) // 4096) * 4096
    Np = ((n + 1 + 255) // 256) * 256
    ncol = 64
    srcp = jnp.concatenate(
        [edge_idx[0].astype(I32), jnp.full((Epad - e,), n, I32)])
    dstp = jnp.concatenate(
        [edge_idx[1].astype(I32), jnp.full((Epad - e,), n, I32)])

    deg2 = _sc_deg(dstp, Epad, Np)
    d0 = deg2[0, :n].reshape(n, 1)
    d1 = deg2[1, :n].reshape(n, 1)

    for p in layers:
        tq = _pad_split(x, Np, ncol)
        acc = _sc_edge_agg(tq, srcp, dstp, None, Epad, Np, ncol)
        agg = jnp.concatenate([acc[q, :n] for q in range(D // ncol)], axis=1)
        y = _sage_post(n)(agg, d0, d1, x, p['Wn'], p['bn'].reshape(1, D),
                          p['Ws'])
        h, alpha, beta = _gat_pre(n)(y, p['Wg'], p['asrc'].reshape(D, 1),
                                     p['adst'].reshape(D, 1))
        cmax, eloop = _gat_scalar(n)(alpha, beta)
        ee, den = _sc_gat_edge(srcp, dstp, _pad1(alpha, Np), _pad1(beta, Np),
                               _pad1(cmax, Np), Epad, Np)
        hq = _pad_split(h, Np, ncol)
        accn = _sc_edge_agg(hq, srcp, dstp, ee, Epad, Np, ncol)
        numer = jnp.concatenate([accn[q, :n] for q in range(D // ncol)],
                                axis=1)
        x = _gat_post(n)(numer, den[0, :n].reshape(n, 1),
                         den[1, :n].reshape(n, 1), eloop, h, y, p['Wr'],
                         p['bg'].reshape(1, D))
    return x


N_DRUG_ROWS = 2000


def kernel(x, edge_idx, x_drug, edge_idx_drug, x_cir, edge_idx_cir, params):
    eh = _branch(x, edge_idx, params['heter'])
    ed = _branch(x_drug, edge_idx_drug, params['drug'])
    ec = _branch(x_cir, edge_idx_cir, params['cir'])
    Rc = jnp.concatenate([eh[:N_DRUG_ROWS], ed], axis=1)
    Dc = jnp.concatenate([eh[N_DRUG_ROWS:], ec], axis=1)
    K2 = Rc.shape[1]
    T = _dec1(Rc.shape[0], K2)(Rc, params['Wdec'])
    return _dec2(T.shape[0], Dc.shape[0], K2)(T, Dc)
